# baseline (speedup 1.0000x reference)
import functools

import jax
import jax.numpy as jnp
from jax import lax
from jax.experimental import pallas as pl
from jax.experimental.pallas import tpu as pltpu
from jax.experimental.pallas import tpu_sc as plsc

B = 2
N = 2048
P2 = 1024
NN = 16
KS = 12
NA = 12
CIN = 64
COUT = 128

D = NA * CIN
WPP = NA * KS * NN
OPP = NA * KS * CIN
NWORK = 32
PTS_W = P2 // NWORK
CHUNK = 2
NCH = PTS_W // CHUNK
LANES = 16
KH = KS // 2


def _sc_stage(feats_rows, w_flat, idx_flat, sidx_flat, xp_rows):
  mesh = plsc.VectorSubcoreMesh(core_axis_name="c", subcore_axis_name="s")

  @functools.partial(
      pl.kernel,
      mesh=mesh,
      out_type=[
          jax.ShapeDtypeStruct((P2 * NA, KS * CIN), jnp.float32),
          jax.ShapeDtypeStruct((P2, 128), jnp.float32),
      ],
      scratch_types=[
          pltpu.VMEM((PTS_W * NN,), jnp.int32),
          pltpu.VMEM((CHUNK * NN, D), jnp.float32),
          pltpu.VMEM((CHUNK * NN, D), jnp.float32),
          pltpu.VMEM((CHUNK * WPP,), jnp.float32),
          pltpu.VMEM((CHUNK * WPP,), jnp.float32),
          pltpu.VMEM((CHUNK * NA, KS * CIN), jnp.float32),
          pltpu.VMEM((CHUNK * NA, KS * CIN), jnp.float32),
          pltpu.VMEM((PTS_W,), jnp.int32),
          pltpu.VMEM((PTS_W, 128), jnp.float32),
          pltpu.SemaphoreType.DMA,
          pltpu.SemaphoreType.DMA,
          pltpu.SemaphoreType.DMA,
          pltpu.SemaphoreType.DMA,
          pltpu.SemaphoreType.DMA,
          pltpu.SemaphoreType.DMA,
      ],
  )
  def sc_kernel(feats_hbm, w_hbm, idx_hbm, sidx_hbm, xp_hbm,
                x_out_hbm, xp_out_hbm,
                idx_all, rows_v0, rows_v1, w_v0, w_v1, out_v0, out_v1,
                sidx_v, xpg_v,
                gsem0, gsem1, wsem0, wsem1, osem0, osem1):
    wid = lax.axis_index("c") * 16 + lax.axis_index("s")
    base_pt = wid * PTS_W

    pltpu.sync_copy(sidx_hbm.at[pl.ds(base_pt, PTS_W)], sidx_v)
    pltpu.async_copy(xp_hbm.at[sidx_v], xpg_v, gsem0).wait()
    pltpu.sync_copy(xpg_v, xp_out_hbm.at[pl.ds(base_pt, PTS_W)])

    pltpu.sync_copy(idx_hbm.at[pl.ds(base_pt * NN, PTS_W * NN)], idx_all)

    def start_fetch(ci, rows_v, w_v, gsem, wsem):
      pt0 = base_pt + ci * CHUNK
      pltpu.async_copy(
          feats_hbm.at[idx_all.at[pl.ds(ci * CHUNK * NN, CHUNK * NN)]],
          rows_v, gsem)
      pltpu.async_copy(w_hbm.at[pl.ds(pt0 * WPP, CHUNK * WPP)], w_v, wsem)

    def wait_fetch(rows_v, w_v, gsem, wsem):
      pltpu.make_async_copy(feats_hbm.at[idx_all.at[pl.ds(0, CHUNK * NN)]],
                            rows_v, gsem).wait()
      pltpu.make_async_copy(w_hbm.at[pl.ds(0, CHUNK * WPP)], w_v, wsem).wait()

    def compute(rows_v, w_v, out_v):
      @pl.loop(0, CHUNK)
      def _(lp):
        @pl.loop(0, NA)
        def _(a):
          for kh in range(2):
            wbase = (lp * NA + a) * KS * NN + kh * KH * NN
            wv = [w_v[pl.ds(wbase + j * NN, NN)] for j in range(KH)]
            acc = [jnp.zeros((LANES,), jnp.float32)
                   for _ in range(KH * 4)]
            for n in range(NN):
              g = [rows_v[lp * NN + n, pl.ds(a * CIN + cb * LANES, LANES)]
                   for cb in range(4)]
              for j in range(KH):
                ws = wv[j][n]
                for cb in range(4):
                  acc[j * 4 + cb] = acc[j * 4 + cb] + g[cb] * ws
            for j in range(KH):
              kk = kh * KH + j
              for cb in range(4):
                out_v[lp * NA + a, pl.ds(kk * CIN + cb * LANES, LANES)] = (
                    acc[j * 4 + cb])

    def store_out(ci, out_v, osem):
      pt0 = base_pt + ci * CHUNK
      pltpu.async_copy(out_v, x_out_hbm.at[pl.ds(pt0 * NA, CHUNK * NA)], osem)

    def drain_out(out_v, osem):
      pltpu.make_async_copy(out_v, x_out_hbm.at[pl.ds(0, CHUNK * NA)],
                            osem).wait()

    start_fetch(0, rows_v0, w_v0, gsem0, wsem0)

    @pl.loop(0, NCH, step=2)
    def _(ci):
      start_fetch(ci + 1, rows_v1, w_v1, gsem1, wsem1)
      wait_fetch(rows_v0, w_v0, gsem0, wsem0)

      @pl.when(ci > 0)
      def _():
        drain_out(out_v0, osem0)
      compute(rows_v0, w_v0, out_v0)
      store_out(ci, out_v0, osem0)

      @pl.when(ci + 2 < NCH)
      def _():
        start_fetch(ci + 2, rows_v0, w_v0, gsem0, wsem0)
      wait_fetch(rows_v1, w_v1, gsem1, wsem1)

      @pl.when(ci > 0)
      def _():
        drain_out(out_v1, osem1)
      compute(rows_v1, w_v1, out_v1)
      store_out(ci + 1, out_v1, osem1)

    drain_out(out_v0, osem0)
    drain_out(out_v1, osem1)

  return sc_kernel(feats_rows, w_flat, idx_flat, sidx_flat, xp_rows)


def _tc_matmul(x2, w2):
  PA = P2 * NA
  BLK = 1024

  def mm(w_ref, x_ref, o_ref):
    o_ref[...] = lax.dot_general(
        w_ref[...], x_ref[...],
        (((1,), (1,)), ((), ())),
        preferred_element_type=jnp.float32)

  return pl.pallas_call(
      mm,
      grid=(PA // BLK,),
      in_specs=[
          pl.BlockSpec((COUT, KS * CIN), lambda i: (0, 0)),
          pl.BlockSpec((BLK, KS * CIN), lambda i: (i, 0)),
      ],
      out_specs=pl.BlockSpec((COUT, BLK), lambda i: (0, i)),
      out_shape=jax.ShapeDtypeStruct((COUT, PA), jnp.float32),
  )(w2, x2)


def kernel(feats, xyz, pose, inter_w, W, inter_idx, sample_idx):
  idx = inter_idx.astype(jnp.int32)
  sidx = sample_idx.astype(jnp.int32)
  w2 = W.reshape(COUT, CIN, KS).transpose(0, 2, 1).reshape(COUT, KS * CIN)

  outs, xyzs, poses = [], [], []
  for b in range(B):
    feats_rows = feats[b].transpose(1, 2, 0).reshape(N, D)
    xp_rows = jnp.pad(
        jnp.concatenate([xyz[b], pose[b].reshape(N, 9)], axis=1),
        ((0, 0), (0, 116)))
    x_flat, xp_g = _sc_stage(
        feats_rows, inter_w[b].reshape(-1), idx[b].reshape(-1),
        sidx[b], xp_rows)
    outs.append(_tc_matmul(x_flat, w2))
    xyzs.append(xp_g[:, :3])
    poses.append(xp_g[:, 3:12].reshape(P2, 3, 3))

  out = jnp.stack(outs).reshape(B, COUT, P2, NA)
  xyz_out = jnp.stack(xyzs)
  sampled_pose = jnp.stack(poses)
  return (inter_idx, inter_w, sample_idx, xyz_out, out, sampled_pose)


# speedup vs baseline: 1.0376x; 1.0376x over previous
import functools

import jax
import jax.numpy as jnp
from jax import lax
from jax.experimental import pallas as pl
from jax.experimental.pallas import tpu as pltpu
from jax.experimental.pallas import tpu_sc as plsc

B = 2
N = 2048
P2 = 1024
NN = 16
KS = 12
NA = 12
CIN = 64
COUT = 128

D = NA * CIN
WPP = NA * KS * NN
NWORK = 32
PTS_W = (B * P2) // NWORK
CHUNK = 2
NCH = PTS_W // CHUNK
LANES = 16
KH = KS // 2


def _sc_stage(feats_rows, w_flat, idx_flat, sidx_flat, xp_rows):
  mesh = plsc.VectorSubcoreMesh(core_axis_name="c", subcore_axis_name="s")

  @functools.partial(
      pl.kernel,
      mesh=mesh,
      out_type=[
          jax.ShapeDtypeStruct((B * P2 * NA, KS * CIN), jnp.float32),
          jax.ShapeDtypeStruct((B * P2, 128), jnp.float32),
      ],
      scratch_types=[
          pltpu.VMEM((PTS_W * NN,), jnp.int32),
          pltpu.VMEM((CHUNK * NN, D), jnp.float32),
          pltpu.VMEM((CHUNK * NN, D), jnp.float32),
          pltpu.VMEM((CHUNK * WPP,), jnp.float32),
          pltpu.VMEM((CHUNK * WPP,), jnp.float32),
          pltpu.VMEM((CHUNK * NA, KS * CIN), jnp.float32),
          pltpu.VMEM((CHUNK * NA, KS * CIN), jnp.float32),
          pltpu.VMEM((PTS_W,), jnp.int32),
          pltpu.VMEM((PTS_W, 128), jnp.float32),
          pltpu.SemaphoreType.DMA,
          pltpu.SemaphoreType.DMA,
          pltpu.SemaphoreType.DMA,
          pltpu.SemaphoreType.DMA,
          pltpu.SemaphoreType.DMA,
          pltpu.SemaphoreType.DMA,
      ],
  )
  def sc_kernel(feats_hbm, w_hbm, idx_hbm, sidx_hbm, xp_hbm,
                x_out_hbm, xp_out_hbm,
                idx_all, rows_v0, rows_v1, w_v0, w_v1, out_v0, out_v1,
                sidx_v, xpg_v,
                gsem0, gsem1, wsem0, wsem1, osem0, osem1):
    wid = lax.axis_index("c") * 16 + lax.axis_index("s")
    base_pt = wid * PTS_W

    pltpu.sync_copy(sidx_hbm.at[pl.ds(base_pt, PTS_W)], sidx_v)
    pltpu.async_copy(xp_hbm.at[sidx_v], xpg_v, gsem0).wait()
    pltpu.sync_copy(xpg_v, xp_out_hbm.at[pl.ds(base_pt, PTS_W)])

    pltpu.sync_copy(idx_hbm.at[pl.ds(base_pt * NN, PTS_W * NN)], idx_all)

    def start_fetch(ci, rows_v, w_v, gsem, wsem):
      pt0 = base_pt + ci * CHUNK
      pltpu.async_copy(
          feats_hbm.at[idx_all.at[pl.ds(ci * CHUNK * NN, CHUNK * NN)]],
          rows_v, gsem)
      pltpu.async_copy(w_hbm.at[pl.ds(pt0 * WPP, CHUNK * WPP)], w_v, wsem)

    def wait_fetch(rows_v, w_v, gsem, wsem):
      pltpu.make_async_copy(feats_hbm.at[idx_all.at[pl.ds(0, CHUNK * NN)]],
                            rows_v, gsem).wait()
      pltpu.make_async_copy(w_hbm.at[pl.ds(0, CHUNK * WPP)], w_v, wsem).wait()

    def compute(rows_v, w_v, out_v):
      @pl.loop(0, CHUNK)
      def _(lp):
        @pl.loop(0, NA)
        def _(a):
          for kh in range(2):
            wbase = (lp * NA + a) * KS * NN + kh * KH * NN
            wv = [w_v[pl.ds(wbase + j * NN, NN)] for j in range(KH)]
            acc = [jnp.zeros((LANES,), jnp.float32)
                   for _ in range(KH * 4)]
            for n in range(NN):
              g = [rows_v[lp * NN + n, pl.ds(a * CIN + cb * LANES, LANES)]
                   for cb in range(4)]
              for j in range(KH):
                ws = wv[j][n]
                for cb in range(4):
                  acc[j * 4 + cb] = acc[j * 4 + cb] + g[cb] * ws
            for j in range(KH):
              kk = kh * KH + j
              for cb in range(4):
                out_v[lp * NA + a, pl.ds(kk * CIN + cb * LANES, LANES)] = (
                    acc[j * 4 + cb])

    def store_out(ci, out_v, osem):
      pt0 = base_pt + ci * CHUNK
      pltpu.async_copy(out_v, x_out_hbm.at[pl.ds(pt0 * NA, CHUNK * NA)], osem)

    def drain_out(out_v, osem):
      pltpu.make_async_copy(out_v, x_out_hbm.at[pl.ds(0, CHUNK * NA)],
                            osem).wait()

    start_fetch(0, rows_v0, w_v0, gsem0, wsem0)

    @pl.loop(0, NCH, step=2)
    def _(ci):
      start_fetch(ci + 1, rows_v1, w_v1, gsem1, wsem1)
      wait_fetch(rows_v0, w_v0, gsem0, wsem0)

      @pl.when(ci > 0)
      def _():
        drain_out(out_v0, osem0)
      compute(rows_v0, w_v0, out_v0)
      store_out(ci, out_v0, osem0)

      @pl.when(ci + 2 < NCH)
      def _():
        start_fetch(ci + 2, rows_v0, w_v0, gsem0, wsem0)
      wait_fetch(rows_v1, w_v1, gsem1, wsem1)

      @pl.when(ci > 0)
      def _():
        drain_out(out_v1, osem1)
      compute(rows_v1, w_v1, out_v1)
      store_out(ci + 1, out_v1, osem1)

    drain_out(out_v0, osem0)
    drain_out(out_v1, osem1)

  return sc_kernel(feats_rows, w_flat, idx_flat, sidx_flat, xp_rows)


def _tc_matmul(x2, w2):
  PA = P2 * NA
  BLK = 1024

  def mm(w_ref, x_ref, o_ref):
    o_ref[...] = lax.dot_general(
        w_ref[...], x_ref[...],
        (((1,), (1,)), ((), ())),
        preferred_element_type=jnp.float32)

  return pl.pallas_call(
      mm,
      grid=(B, PA // BLK),
      in_specs=[
          pl.BlockSpec((COUT, KS * CIN), lambda b, i: (0, 0)),
          pl.BlockSpec((None, BLK, KS * CIN), lambda b, i: (b, i, 0)),
      ],
      out_specs=pl.BlockSpec((None, COUT, BLK), lambda b, i: (b, 0, i)),
      out_shape=jax.ShapeDtypeStruct((B, COUT, PA), jnp.float32),
  )(w2, x2)


def kernel(feats, xyz, pose, inter_w, W, inter_idx, sample_idx):
  idx = inter_idx.astype(jnp.int32)
  sidx = sample_idx.astype(jnp.int32)

  feats_rows = feats.transpose(0, 2, 3, 1).reshape(B * N, D)
  off = jnp.arange(B, dtype=jnp.int32) * N
  idx_flat = (idx + off[:, None, None]).reshape(-1)
  sidx_flat = (sidx + off[:, None]).reshape(-1)
  w_flat = inter_w.reshape(-1)
  xp_rows = jnp.pad(
      jnp.concatenate([xyz.reshape(B * N, 3), pose.reshape(B * N, 9)], axis=1),
      ((0, 0), (0, 116)))

  x_flat, xp_g = _sc_stage(
      feats_rows, w_flat, idx_flat, sidx_flat, xp_rows)

  x2 = x_flat.reshape(B, P2 * NA, KS * CIN)
  w2 = W.reshape(COUT, CIN, KS).transpose(0, 2, 1).reshape(COUT, KS * CIN)
  out = _tc_matmul(x2, w2).reshape(B, COUT, P2, NA)

  xyz_out = xp_g[:, :3].reshape(B, P2, 3)
  sampled_pose = xp_g[:, 3:12].reshape(B, P2, 3, 3)
  return (inter_idx, inter_w, sample_idx, xyz_out, out, sampled_pose)
